# Initial kernel scaffold; baseline (speedup 1.0000x reference)
import functools

import jax
import jax.numpy as jnp
from jax import lax
from jax.experimental import pallas as pl
from jax.experimental.pallas import tpu as pltpu
from jax.experimental.pallas import tpu_sc as plsc

N = 10000
E = 320000
D = 128
NOISE = 64
HID = 256
NARG = 4
B = 500
NPG = 20

NC = 2
NS = 16
NW = NC * NS
K = 128
CPT = 80
NCH = NW * CPT
EP = NCH * K
PAD_ROWS = 16
NPAD = N + PAD_ROWS
RPT = NPAD // NS

_sc_mesh = plsc.VectorSubcoreMesh(
    core_axis_name="c", subcore_axis_name="s", num_cores=NC, num_subcores=NS)


@functools.partial(
    pl.kernel,
    mesh=_sc_mesh,
    out_type=jax.ShapeDtypeStruct((NC, NPAD, D), jnp.float32),
    scratch_types=[
        pltpu.VMEM((CPT, K), jnp.int32),
        pltpu.VMEM((CPT, K), jnp.int32),
        pltpu.VMEM((K, D), jnp.float32),
        pltpu.VMEM((K, D), jnp.float32),
        pltpu.VMEM_SHARED((NPAD, D), jnp.float32),
        pltpu.SemaphoreType.DMA,
        pltpu.SemaphoreType.DMA,
    ],
)
def _sc_agg(xsel_hbm, src_hbm, dst_hbm, zeros_hbm, out_hbm,
            src_v, dst_v, rows0, rows1, agg_sh, sem0, sem1):
    c = lax.axis_index("c")
    s = lax.axis_index("s")
    wid = s * NC + c

    pltpu.sync_copy(zeros_hbm, agg_sh.at[pl.ds(s * RPT, RPT)])
    plsc.subcore_barrier()

    base = wid * CPT
    pltpu.sync_copy(src_hbm.at[pl.ds(base, CPT)], src_v)
    pltpu.sync_copy(dst_hbm.at[pl.ds(base, CPT)], dst_v)

    def fire(j, rows, sem):
        pltpu.async_copy(xsel_hbm.at[src_v.at[j]], rows, sem)

    def drain(rows, sem):
        pltpu.make_async_copy(xsel_hbm.at[pl.ds(0, K)], rows, sem).wait()

    def scat(j, rows):
        pltpu.sync_copy(rows, agg_sh.at[dst_v.at[j]], add=True)

    fire(0, rows0, sem0)

    def body(i, carry):
        j = i * 2
        fire(j + 1, rows1, sem1)
        drain(rows0, sem0)
        scat(j, rows0)

        @pl.when(j + 2 < CPT)
        def _():
            fire(j + 2, rows0, sem0)

        drain(rows1, sem1)
        scat(j + 1, rows1)
        return carry

    lax.fori_loop(0, CPT // 2, body, 0)

    plsc.subcore_barrier()
    pltpu.sync_copy(agg_sh.at[pl.ds(s * RPT, RPT)],
                    out_hbm.at[c, pl.ds(s * RPT, RPT)])


def _tc_body(xp_ref, oabb_ref, noise_ref, aggp_ref, Ws_ref, Wn_ref, bg_ref,
             W1a_ref, W1b_ref, W1n_ref, b1_ref, W2_ref, b2_ref, out_ref):
    f32 = jnp.float32
    xb = xp_ref[0]
    agg = aggp_ref[0, 0] + aggp_ref[1, 0]
    ox = jnp.dot(xb, Ws_ref[...], preferred_element_type=f32)
    ox += jnp.dot(agg, Wn_ref[...], preferred_element_type=f32)
    ox = jnp.maximum(ox + bg_ref[...], 0.0)
    h = jnp.dot(ox, W1a_ref[0], preferred_element_type=f32)
    h += jnp.dot(oabb_ref[0], W1b_ref[0], preferred_element_type=f32)
    h += jnp.dot(noise_ref[...], W1n_ref[0], preferred_element_type=f32)
    h = jnp.maximum(h + b1_ref[0], 0.0)
    out_ref[0] = jnp.dot(h, W2_ref[0], preferred_element_type=f32) + b2_ref[0]


_tc_mlp = pl.pallas_call(
    _tc_body,
    grid=(NPG,),
    in_specs=[
        pl.BlockSpec((1, B, D), lambda p: (p, 0, 0)),
        pl.BlockSpec((1, B, 6), lambda p: (p, 0, 0)),
        pl.BlockSpec((B, NOISE), lambda p: (0, 0)),
        pl.BlockSpec((NC, 1, B, D), lambda p: (0, p, 0, 0)),
        pl.BlockSpec((D, D), lambda p: (0, 0)),
        pl.BlockSpec((D, D), lambda p: (0, 0)),
        pl.BlockSpec((1, D), lambda p: (0, 0)),
        pl.BlockSpec((1, D, HID), lambda p: (p, 0, 0)),
        pl.BlockSpec((1, 6, HID), lambda p: (p, 0, 0)),
        pl.BlockSpec((1, NOISE, HID), lambda p: (p, 0, 0)),
        pl.BlockSpec((1, 1, HID), lambda p: (p, 0, 0)),
        pl.BlockSpec((1, HID, D), lambda p: (p, 0, 0)),
        pl.BlockSpec((1, 1, D), lambda p: (p, 0, 0)),
    ],
    out_specs=pl.BlockSpec((1, B, D), lambda p: (p, 0, 0)),
    out_shape=jax.ShapeDtypeStruct((NPG, B, D), jnp.float32),
)


def kernel(x, edge_index, ptr, noise, Ws, Wn, bg, aW1, ab1, aW2, ab2,
           sW1, sb1, sW2, sb2):
    del ptr
    xsel = x[:, :D]
    xp = xsel.reshape(B, NPG, D).transpose(1, 0, 2)
    oabbp = x[:, 13:19].reshape(B, NPG, 6).transpose(1, 0, 2)

    src = edge_index[0]
    dst = edge_index[1]
    dstr = (dst % NPG) * B + dst // NPG
    npad = EP - E
    pad_iota = jnp.arange(npad, dtype=jnp.int32)
    srcp = jnp.concatenate([src, pad_iota % N]).reshape(NCH, K)
    dstp = jnp.concatenate([dstr, N + pad_iota % PAD_ROWS]).reshape(NCH, K)
    zeros = jnp.zeros((RPT, D), jnp.float32)

    aggp = _sc_agg(xsel, srcp, dstp, zeros)
    aggp = aggp[:, :N, :].reshape(NC, NPG, B, D)

    nsur = NPG - NARG
    W1a = jnp.concatenate([aW1[:, :D, :],
                           jnp.tile(sW1[None, :D, :], (nsur, 1, 1))])
    W1b = jnp.concatenate([aW1[:, D:D + 6, :],
                           jnp.tile(sW1[None, D:D + 6, :], (nsur, 1, 1))])
    W1n = jnp.concatenate([aW1[:, D + 6:, :],
                           jnp.tile(sW1[None, D + 6:, :], (nsur, 1, 1))])
    b1 = jnp.concatenate([ab1, jnp.tile(sb1[None, :], (nsur, 1))])
    W2 = jnp.concatenate([aW2, jnp.tile(sW2[None, :, :], (nsur, 1, 1))])
    b2 = jnp.concatenate([ab2, jnp.tile(sb2[None, :], (nsur, 1))])

    outp = _tc_mlp(xp, oabbp, noise, aggp, Ws, Wn, bg.reshape(1, D),
                   W1a, W1b, W1n, b1.reshape(NPG, 1, HID),
                   W2, b2.reshape(NPG, 1, D))
    return outp.transpose(1, 0, 2).reshape(N, D)


# speedup vs baseline: 186.7166x; 186.7166x over previous
import functools

import jax
import jax.numpy as jnp
from jax import lax
from jax.experimental import pallas as pl
from jax.experimental.pallas import tpu as pltpu
from jax.experimental.pallas import tpu_sc as plsc

N = 10000
E = 320000
D = 128
NOISE = 64
HID = 256
NARG = 4
B = 500
NPG = 20

NC = 2
NS = 16
NW = NC * NS
K = 128
CPT = 80
BC = 16
NB = CPT // BC
NCH = NW * CPT
EP = NCH * K
PAD_ROWS = 112
NPAD = N + PAD_ROWS
RPT = NPAD // NS

@functools.lru_cache(maxsize=1)
def _make_sc_agg():
  mesh = plsc.VectorSubcoreMesh(
      core_axis_name="c", subcore_axis_name="s", num_cores=NC, num_subcores=NS)

  @functools.partial(
      pl.kernel,
      mesh=mesh,
      out_type=jax.ShapeDtypeStruct((NC, NPAD, D), jnp.float32),
      scratch_types=[
          pltpu.VMEM((BC, K), jnp.int32),
          pltpu.VMEM((BC, K), jnp.int32),
          pltpu.VMEM((K, D), jnp.float32),
          pltpu.VMEM((K, D), jnp.float32),
          pltpu.VMEM_SHARED((NPAD, D), jnp.float32),
          pltpu.SemaphoreType.DMA,
          pltpu.SemaphoreType.DMA,
      ],
  )
  def _sc_agg(xsel_hbm, src_hbm, dst_hbm, zeros_hbm, out_hbm,
              src_v, dst_v, rows0, rows1, agg_sh, sem0, sem1):
    c = lax.axis_index("c")
    s = lax.axis_index("s")
    wid = s * NC + c

    pltpu.sync_copy(zeros_hbm, agg_sh.at[pl.ds(s * RPT, RPT)])
    plsc.subcore_barrier()

    base = wid * CPT

    def fire(j, rows, sem):
        pltpu.async_copy(xsel_hbm.at[src_v.at[j]], rows, sem)

    def drain(rows, sem):
        pltpu.make_async_copy(xsel_hbm.at[pl.ds(0, K)], rows, sem).wait()

    def scat(j, rows):
        pltpu.sync_copy(rows, agg_sh.at[dst_v.at[j]], add=True)

    def batch(b, carry):
        pltpu.sync_copy(src_hbm.at[pl.ds(base + b * BC, BC)], src_v)
        pltpu.sync_copy(dst_hbm.at[pl.ds(base + b * BC, BC)], dst_v)
        fire(0, rows0, sem0)

        def body(i, carry2):
            j = i * 2
            fire(j + 1, rows1, sem1)
            drain(rows0, sem0)
            scat(j, rows0)

            @pl.when(j + 2 < BC)
            def _():
                fire(j + 2, rows0, sem0)

            drain(rows1, sem1)
            scat(j + 1, rows1)
            return carry2

        lax.fori_loop(0, BC // 2, body, 0)
        return carry

    lax.fori_loop(0, NB, batch, 0)

    plsc.subcore_barrier()
    pltpu.sync_copy(agg_sh.at[pl.ds(s * RPT, RPT)],
                    out_hbm.at[c, pl.ds(s * RPT, RPT)])

  return _sc_agg


def _tc_body(xp_ref, oabb_ref, noise_ref, aggp_ref, Ws_ref, Wn_ref, bg_ref,
             W1a_ref, W1b_ref, W1n_ref, b1_ref, W2_ref, b2_ref, out_ref):
    f32 = jnp.float32
    xb = xp_ref[0]
    agg = aggp_ref[0, 0] + aggp_ref[1, 0]
    ox = jnp.dot(xb, Ws_ref[...], preferred_element_type=f32)
    ox += jnp.dot(agg, Wn_ref[...], preferred_element_type=f32)
    ox = jnp.maximum(ox + bg_ref[...], 0.0)
    h = jnp.dot(ox, W1a_ref[0], preferred_element_type=f32)
    h += jnp.dot(oabb_ref[0], W1b_ref[0], preferred_element_type=f32)
    h += jnp.dot(noise_ref[...], W1n_ref[0], preferred_element_type=f32)
    h = jnp.maximum(h + b1_ref[0], 0.0)
    out_ref[0] = jnp.dot(h, W2_ref[0], preferred_element_type=f32) + b2_ref[0]


_tc_mlp = pl.pallas_call(
    _tc_body,
    grid=(NPG,),
    in_specs=[
        pl.BlockSpec((1, B, D), lambda p: (p, 0, 0)),
        pl.BlockSpec((1, B, 6), lambda p: (p, 0, 0)),
        pl.BlockSpec((B, NOISE), lambda p: (0, 0)),
        pl.BlockSpec((NC, 1, B, D), lambda p: (0, p, 0, 0)),
        pl.BlockSpec((D, D), lambda p: (0, 0)),
        pl.BlockSpec((D, D), lambda p: (0, 0)),
        pl.BlockSpec((1, D), lambda p: (0, 0)),
        pl.BlockSpec((1, D, HID), lambda p: (p, 0, 0)),
        pl.BlockSpec((1, 6, HID), lambda p: (p, 0, 0)),
        pl.BlockSpec((1, NOISE, HID), lambda p: (p, 0, 0)),
        pl.BlockSpec((1, 1, HID), lambda p: (p, 0, 0)),
        pl.BlockSpec((1, HID, D), lambda p: (p, 0, 0)),
        pl.BlockSpec((1, 1, D), lambda p: (p, 0, 0)),
    ],
    out_specs=pl.BlockSpec((1, B, D), lambda p: (p, 0, 0)),
    out_shape=jax.ShapeDtypeStruct((NPG, B, D), jnp.float32),
)


def kernel(x, edge_index, ptr, noise, Ws, Wn, bg, aW1, ab1, aW2, ab2,
           sW1, sb1, sW2, sb2):
    del ptr
    xsel = x[:, :D]
    xp = xsel.reshape(B, NPG, D).transpose(1, 0, 2)
    oabbp = x[:, 13:19].reshape(B, NPG, 6).transpose(1, 0, 2)

    src = edge_index[0]
    dst = edge_index[1]
    dstr = (dst % NPG) * B + dst // NPG
    npad = EP - E
    pad_iota = jnp.arange(npad, dtype=jnp.int32)
    srcp = jnp.concatenate([src, pad_iota % N]).reshape(NCH, K)
    dstp = jnp.concatenate([dstr, N + pad_iota % PAD_ROWS]).reshape(NCH, K)
    zeros = jnp.zeros((RPT, D), jnp.float32)

    aggp = _make_sc_agg()(xsel, srcp, dstp, zeros)
    aggp = aggp[:, :N, :].reshape(NC, NPG, B, D)

    nsur = NPG - NARG
    W1a = jnp.concatenate([aW1[:, :D, :],
                           jnp.tile(sW1[None, :D, :], (nsur, 1, 1))])
    W1b = jnp.concatenate([aW1[:, D:D + 6, :],
                           jnp.tile(sW1[None, D:D + 6, :], (nsur, 1, 1))])
    W1n = jnp.concatenate([aW1[:, D + 6:, :],
                           jnp.tile(sW1[None, D + 6:, :], (nsur, 1, 1))])
    b1 = jnp.concatenate([ab1, jnp.tile(sb1[None, :], (nsur, 1))])
    W2 = jnp.concatenate([aW2, jnp.tile(sW2[None, :, :], (nsur, 1, 1))])
    b2 = jnp.concatenate([ab2, jnp.tile(sb2[None, :], (nsur, 1))])

    outp = _tc_mlp(xp, oabbp, noise, aggp, Ws, Wn, bg.reshape(1, D),
                   W1a, W1b, W1n, b1.reshape(NPG, 1, HID),
                   W2, b2.reshape(NPG, 1, D))
    return outp.transpose(1, 0, 2).reshape(N, D)


# speedup vs baseline: 195.1041x; 1.0449x over previous
import functools

import jax
import jax.numpy as jnp
from jax import lax
from jax.experimental import pallas as pl
from jax.experimental.pallas import tpu as pltpu
from jax.experimental.pallas import tpu_sc as plsc

N = 10000
E = 320000
D = 128
NOISE = 64
HID = 256
NARG = 4
B = 500
NPG = 20

NC = 2
NS = 16
NW = NC * NS
K = 128
CPT = 80
BC = 40
NB = CPT // BC
NCH = NW * CPT
EP = NCH * K
PAD_ROWS = 112
NPAD = N + PAD_ROWS
RPT = NPAD // NS

@functools.lru_cache(maxsize=1)
def _make_sc_agg():
  mesh = plsc.VectorSubcoreMesh(
      core_axis_name="c", subcore_axis_name="s", num_cores=NC, num_subcores=NS)

  @functools.partial(
      pl.kernel,
      mesh=mesh,
      out_type=jax.ShapeDtypeStruct((NC, NPAD, D), jnp.float32),
      scratch_types=[
          pltpu.VMEM((BC, K), jnp.int32),
          pltpu.VMEM((BC, K), jnp.int32),
          pltpu.VMEM((K, D), jnp.float32),
          pltpu.VMEM((K, D), jnp.float32),
          pltpu.VMEM_SHARED((NPAD, D), jnp.float32),
          pltpu.SemaphoreType.DMA,
          pltpu.SemaphoreType.DMA,
      ],
  )
  def _sc_agg(xsel_hbm, src_hbm, dst_hbm, zeros_hbm, out_hbm,
              src_v, dst_v, rows0, rows1, agg_sh, sem0, sem1):
    c = lax.axis_index("c")
    s = lax.axis_index("s")
    wid = s * NC + c

    pltpu.sync_copy(zeros_hbm, agg_sh.at[pl.ds(s * RPT, RPT)])
    plsc.subcore_barrier()

    base = wid * CPT

    def fire(j, rows, sem):
        pltpu.async_copy(xsel_hbm.at[src_v.at[j]], rows, sem)

    def drain(rows, sem):
        pltpu.make_async_copy(xsel_hbm.at[pl.ds(0, K)], rows, sem).wait()

    def scat(j, rows):
        pltpu.sync_copy(rows, agg_sh.at[dst_v.at[j]], add=True)

    def batch(b, carry):
        pltpu.sync_copy(src_hbm.at[pl.ds(base + b * BC, BC)], src_v)
        pltpu.sync_copy(dst_hbm.at[pl.ds(base + b * BC, BC)], dst_v)
        fire(0, rows0, sem0)

        def body(i, carry2):
            j = i * 2
            fire(j + 1, rows1, sem1)
            drain(rows0, sem0)
            scat(j, rows0)

            @pl.when(j + 2 < BC)
            def _():
                fire(j + 2, rows0, sem0)

            drain(rows1, sem1)
            scat(j + 1, rows1)
            return carry2

        lax.fori_loop(0, BC // 2, body, 0)
        return carry

    lax.fori_loop(0, NB, batch, 0)

    plsc.subcore_barrier()
    pltpu.sync_copy(agg_sh.at[pl.ds(s * RPT, RPT)],
                    out_hbm.at[c, pl.ds(s * RPT, RPT)])

  return _sc_agg


def _tc_body(xp_ref, oabb_ref, noise_ref, aggp_ref, Ws_ref, Wn_ref, bg_ref,
             W1a_ref, W1b_ref, W1n_ref, b1_ref, W2_ref, b2_ref, out_ref):
    f32 = jnp.float32
    xb = xp_ref[0]
    agg = aggp_ref[0, 0] + aggp_ref[1, 0]
    ox = jnp.dot(xb, Ws_ref[...], preferred_element_type=f32)
    ox += jnp.dot(agg, Wn_ref[...], preferred_element_type=f32)
    ox = jnp.maximum(ox + bg_ref[...], 0.0)
    h = jnp.dot(ox, W1a_ref[0], preferred_element_type=f32)
    h += jnp.dot(oabb_ref[0], W1b_ref[0], preferred_element_type=f32)
    h += jnp.dot(noise_ref[...], W1n_ref[0], preferred_element_type=f32)
    h = jnp.maximum(h + b1_ref[0], 0.0)
    out_ref[0] = jnp.dot(h, W2_ref[0], preferred_element_type=f32) + b2_ref[0]


_tc_mlp = pl.pallas_call(
    _tc_body,
    grid=(NPG,),
    in_specs=[
        pl.BlockSpec((1, B, D), lambda p: (p, 0, 0)),
        pl.BlockSpec((1, B, 6), lambda p: (p, 0, 0)),
        pl.BlockSpec((B, NOISE), lambda p: (0, 0)),
        pl.BlockSpec((NC, 1, B, D), lambda p: (0, p, 0, 0)),
        pl.BlockSpec((D, D), lambda p: (0, 0)),
        pl.BlockSpec((D, D), lambda p: (0, 0)),
        pl.BlockSpec((1, D), lambda p: (0, 0)),
        pl.BlockSpec((1, D, HID), lambda p: (jnp.minimum(p, NARG), 0, 0)),
        pl.BlockSpec((1, 6, HID), lambda p: (jnp.minimum(p, NARG), 0, 0)),
        pl.BlockSpec((1, NOISE, HID), lambda p: (jnp.minimum(p, NARG), 0, 0)),
        pl.BlockSpec((1, 1, HID), lambda p: (jnp.minimum(p, NARG), 0, 0)),
        pl.BlockSpec((1, HID, D), lambda p: (jnp.minimum(p, NARG), 0, 0)),
        pl.BlockSpec((1, 1, D), lambda p: (jnp.minimum(p, NARG), 0, 0)),
    ],
    out_specs=pl.BlockSpec((1, B, D), lambda p: (p, 0, 0)),
    out_shape=jax.ShapeDtypeStruct((NPG, B, D), jnp.float32),
)


def kernel(x, edge_index, ptr, noise, Ws, Wn, bg, aW1, ab1, aW2, ab2,
           sW1, sb1, sW2, sb2):
    del ptr
    xsel = x[:, :D]
    xp = xsel.reshape(B, NPG, D).transpose(1, 0, 2)
    oabbp = x[:, 13:19].reshape(B, NPG, 6).transpose(1, 0, 2)

    src = edge_index[0]
    dst = edge_index[1]
    dstr = (dst % NPG) * B + dst // NPG
    npad = EP - E
    pad_iota = jnp.arange(npad, dtype=jnp.int32)
    srcp = jnp.concatenate([src, pad_iota % N]).reshape(NCH, K)
    dstp = jnp.concatenate([dstr, N + pad_iota % PAD_ROWS]).reshape(NCH, K)
    zeros = jnp.zeros((RPT, D), jnp.float32)

    aggp = _make_sc_agg()(xsel, srcp, dstp, zeros)
    aggp = aggp[:, :N, :].reshape(NC, NPG, B, D)

    NU = NARG + 1
    W1 = jnp.concatenate([aW1, sW1[None]])
    W1a = W1[:, :D, :]
    W1b = W1[:, D:D + 6, :]
    W1n = W1[:, D + 6:, :]
    b1 = jnp.concatenate([ab1, sb1[None]])
    W2 = jnp.concatenate([aW2, sW2[None]])
    b2 = jnp.concatenate([ab2, sb2[None]])

    outp = _tc_mlp(xp, oabbp, noise, aggp, Ws, Wn, bg.reshape(1, D),
                   W1a, W1b, W1n, b1.reshape(NU, 1, HID),
                   W2, b2.reshape(NU, 1, D))
    return outp.transpose(1, 0, 2).reshape(N, D)


# speedup vs baseline: 212.6108x; 1.0897x over previous
import functools

import jax
import jax.numpy as jnp
from jax import lax
from jax.experimental import pallas as pl
from jax.experimental.pallas import tpu as pltpu
from jax.experimental.pallas import tpu_sc as plsc

N = 10000
E = 320000
D = 128
NOISE = 64
HID = 256
NARG = 4
B = 500
NPG = 20

NC = 2
NS = 16
NW = NC * NS
K = 128
CPT = 80
BC = 40
NB = CPT // BC
NCH = NW * CPT
EP = NCH * K
B2 = 512
NPAD = NPG * B2
RPT = NPAD // NS

@functools.lru_cache(maxsize=1)
def _make_sc_agg():
  mesh = plsc.VectorSubcoreMesh(
      core_axis_name="c", subcore_axis_name="s", num_cores=NC, num_subcores=NS)

  @functools.partial(
      pl.kernel,
      mesh=mesh,
      out_type=jax.ShapeDtypeStruct((NC, NPAD, D), jnp.float32),
      scratch_types=[
          pltpu.VMEM((BC, K), jnp.int32),
          pltpu.VMEM((BC, K), jnp.int32),
          pltpu.VMEM((K, D), jnp.float32),
          pltpu.VMEM((K, D), jnp.float32),
          pltpu.VMEM_SHARED((NPAD, D), jnp.float32),
          pltpu.SemaphoreType.DMA,
          pltpu.SemaphoreType.DMA,
      ],
  )
  def _sc_agg(xsel_hbm, src_hbm, dst_hbm, zeros_hbm, out_hbm,
              src_v, dst_v, rows0, rows1, agg_sh, sem0, sem1):
    c = lax.axis_index("c")
    s = lax.axis_index("s")
    wid = s * NC + c

    pltpu.sync_copy(zeros_hbm, agg_sh.at[pl.ds(s * RPT, RPT)])
    plsc.subcore_barrier()

    base = wid * CPT

    def fire(j, rows, sem):
        pltpu.async_copy(xsel_hbm.at[src_v.at[j]], rows, sem)

    def drain(rows, sem):
        pltpu.make_async_copy(xsel_hbm.at[pl.ds(0, K)], rows, sem).wait()

    def scat(j, rows):
        pltpu.sync_copy(rows, agg_sh.at[dst_v.at[j]], add=True)

    def batch(b, carry):
        pltpu.sync_copy(src_hbm.at[pl.ds(base + b * BC, BC)], src_v)
        pltpu.sync_copy(dst_hbm.at[pl.ds(base + b * BC, BC)], dst_v)
        fire(0, rows0, sem0)

        def body(i, carry2):
            j = i * 2
            fire(j + 1, rows1, sem1)
            drain(rows0, sem0)
            scat(j, rows0)

            @pl.when(j + 2 < BC)
            def _():
                fire(j + 2, rows0, sem0)

            drain(rows1, sem1)
            scat(j + 1, rows1)
            return carry2

        lax.fori_loop(0, BC // 2, body, 0)
        return carry

    lax.fori_loop(0, NB, batch, 0)

    plsc.subcore_barrier()
    pltpu.sync_copy(agg_sh.at[pl.ds(s * RPT, RPT)],
                    out_hbm.at[c, pl.ds(s * RPT, RPT)])

  return _sc_agg


def _tc_body(xp_ref, noise_ref, aggp_ref, Ws_ref, Wn_ref, bg_ref,
             W1a_ref, W1b_ref, W1n_ref, b1_ref, W2_ref, b2_ref, out_ref):
    f32 = jnp.float32
    xb = xp_ref[0]
    agg = aggp_ref[0, 0, :B] + aggp_ref[1, 0, :B]
    ox = jnp.dot(xb, Ws_ref[...], preferred_element_type=f32)
    ox += jnp.dot(agg, Wn_ref[...], preferred_element_type=f32)
    ox = jnp.maximum(ox + bg_ref[...], 0.0)
    h = jnp.dot(ox, W1a_ref[0], preferred_element_type=f32)
    h += jnp.dot(xb[:, 13:19], W1b_ref[0], preferred_element_type=f32)
    h += jnp.dot(noise_ref[...], W1n_ref[0], preferred_element_type=f32)
    h = jnp.maximum(h + b1_ref[0], 0.0)
    out_ref[0] = jnp.dot(h, W2_ref[0], preferred_element_type=f32) + b2_ref[0]


_tc_mlp = pl.pallas_call(
    _tc_body,
    grid=(NPG,),
    in_specs=[
        pl.BlockSpec((1, B, D), lambda p: (p, 0, 0)),
        pl.BlockSpec((B, NOISE), lambda p: (0, 0)),
        pl.BlockSpec((NC, 1, B2, D), lambda p: (0, p, 0, 0)),
        pl.BlockSpec((D, D), lambda p: (0, 0)),
        pl.BlockSpec((D, D), lambda p: (0, 0)),
        pl.BlockSpec((1, D), lambda p: (0, 0)),
        pl.BlockSpec((1, D, HID), lambda p: (jnp.minimum(p, NARG), 0, 0)),
        pl.BlockSpec((1, 6, HID), lambda p: (jnp.minimum(p, NARG), 0, 0)),
        pl.BlockSpec((1, NOISE, HID), lambda p: (jnp.minimum(p, NARG), 0, 0)),
        pl.BlockSpec((1, 1, HID), lambda p: (jnp.minimum(p, NARG), 0, 0)),
        pl.BlockSpec((1, HID, D), lambda p: (jnp.minimum(p, NARG), 0, 0)),
        pl.BlockSpec((1, 1, D), lambda p: (jnp.minimum(p, NARG), 0, 0)),
    ],
    out_specs=pl.BlockSpec((1, B, D), lambda p: (p, 0, 0)),
    out_shape=jax.ShapeDtypeStruct((NPG, B, D), jnp.float32),
)


def kernel(x, edge_index, ptr, noise, Ws, Wn, bg, aW1, ab1, aW2, ab2,
           sW1, sb1, sW2, sb2):
    del ptr
    xsel = x[:, :D]
    xp = xsel.reshape(B, NPG, D).transpose(1, 0, 2)

    src = edge_index[0]
    dst = edge_index[1]
    dstr = (dst % NPG) * B2 + dst // NPG
    npad = EP - E
    pad_iota = jnp.arange(npad, dtype=jnp.int32)
    srcp = jnp.concatenate([src, pad_iota % N]).reshape(NCH, K)
    dstp = jnp.concatenate(
        [dstr, (pad_iota % NPG) * B2 + B + pad_iota % (B2 - B)]).reshape(NCH, K)
    zeros = jnp.zeros((RPT, D), jnp.float32)

    aggp = _make_sc_agg()(xsel, srcp, dstp, zeros)
    aggp = aggp.reshape(NC, NPG, B2, D)

    NU = NARG + 1
    W1 = jnp.concatenate([aW1, sW1[None]])
    W1a = W1[:, :D, :]
    W1b = W1[:, D:D + 6, :]
    W1n = W1[:, D + 6:, :]
    b1 = jnp.concatenate([ab1, sb1[None]])
    W2 = jnp.concatenate([aW2, sW2[None]])
    b2 = jnp.concatenate([ab2, sb2[None]])

    outp = _tc_mlp(xp, noise, aggp, Ws, Wn, bg.reshape(1, D),
                   W1a, W1b, W1n, b1.reshape(NU, 1, HID),
                   W2, b2.reshape(NU, 1, D))
    return outp.transpose(1, 0, 2).reshape(N, D)


# speedup vs baseline: 240.3682x; 1.1306x over previous
import functools

import jax
import jax.numpy as jnp
from jax import lax
from jax.experimental import pallas as pl
from jax.experimental.pallas import tpu as pltpu
from jax.experimental.pallas import tpu_sc as plsc

N = 10000
E = 320000
D = 128
NOISE = 64
HID = 256
NARG = 4
B = 500
NPG = 20

NC = 2
NS = 16
NW = NC * NS
K = 128
CPT = 80
BC = 40
NB = CPT // BC
NCH = NW * CPT
EP = NCH * K
B2 = 512
NPAD = NPG * B2
RPT = NPAD // NS

@functools.lru_cache(maxsize=1)
def _make_sc_agg():
  mesh = plsc.VectorSubcoreMesh(
      core_axis_name="c", subcore_axis_name="s", num_cores=NC, num_subcores=NS)

  @functools.partial(
      pl.kernel,
      mesh=mesh,
      out_type=jax.ShapeDtypeStruct((NC, NPAD, D), jnp.float32),
      scratch_types=[
          pltpu.VMEM((2, BC * K), jnp.int32),
          pltpu.VMEM((K,), jnp.int32),
          pltpu.VMEM((K, D), jnp.float32),
          pltpu.VMEM((K, D), jnp.float32),
          pltpu.VMEM_SHARED((NPAD, D), jnp.float32),
          pltpu.SemaphoreType.DMA,
          pltpu.SemaphoreType.DMA,
      ],
  )
  def _sc_agg(xsel_hbm, ei_hbm, zeros_hbm, out_hbm,
              idx_v, dstr_v, rows0, rows1, agg_sh, sem0, sem1):
    c = lax.axis_index("c")
    s = lax.axis_index("s")
    wid = s * NC + c

    pltpu.sync_copy(zeros_hbm, agg_sh.at[pl.ds(s * RPT, RPT)])
    plsc.subcore_barrier()

    base = wid * CPT * K

    def fire(j, rows, sem):
        pltpu.async_copy(xsel_hbm.at[idx_v.at[0, pl.ds(j * K, K)]], rows, sem)

    def drain(rows, sem):
        pltpu.make_async_copy(xsel_hbm.at[pl.ds(0, K)], rows, sem).wait()

    def scat(j, rows):
        for l in range(K // 16):
            d = idx_v[1, pl.ds(j * K + l * 16, 16)]
            q = lax.shift_right_logical(d * 3277, 16)
            r = d - q * NPG
            dstr_v[pl.ds(l * 16, 16)] = lax.shift_left(r, 9) + q
        pltpu.sync_copy(rows, agg_sh.at[dstr_v], add=True)

    def batch(b, carry):
        pltpu.sync_copy(ei_hbm.at[:, pl.ds(base + b * (BC * K), BC * K)], idx_v)
        fire(0, rows0, sem0)

        def body(i, carry2):
            j = i * 2
            fire(j + 1, rows1, sem1)
            drain(rows0, sem0)
            scat(j, rows0)

            @pl.when(j + 2 < BC)
            def _():
                fire(j + 2, rows0, sem0)

            drain(rows1, sem1)
            scat(j + 1, rows1)
            return carry2

        lax.fori_loop(0, BC // 2, body, 0)
        return carry

    lax.fori_loop(0, NB, batch, 0)

    plsc.subcore_barrier()
    pltpu.sync_copy(agg_sh.at[pl.ds(s * RPT, RPT)],
                    out_hbm.at[c, pl.ds(s * RPT, RPT)])

  return _sc_agg


def _tc_body(xp_ref, noise_ref, aggp_ref, Ws_ref, Wn_ref, bg_ref,
             W1a_ref, W1b_ref, W1n_ref, b1_ref, W2_ref, b2_ref, out_ref):
    f32 = jnp.float32
    xb = xp_ref[0]
    agg = aggp_ref[0, 0, :B] + aggp_ref[1, 0, :B]
    ox = jnp.dot(xb, Ws_ref[...], preferred_element_type=f32)
    ox += jnp.dot(agg, Wn_ref[...], preferred_element_type=f32)
    ox = jnp.maximum(ox + bg_ref[...], 0.0)
    h = jnp.dot(ox, W1a_ref[0], preferred_element_type=f32)
    h += jnp.dot(xb[:, 13:19], W1b_ref[0], preferred_element_type=f32)
    h += jnp.dot(noise_ref[...], W1n_ref[0], preferred_element_type=f32)
    h = jnp.maximum(h + b1_ref[0], 0.0)
    out_ref[0] = jnp.dot(h, W2_ref[0], preferred_element_type=f32) + b2_ref[0]


_tc_mlp = pl.pallas_call(
    _tc_body,
    grid=(NPG,),
    in_specs=[
        pl.BlockSpec((1, B, D), lambda p: (p, 0, 0)),
        pl.BlockSpec((B, NOISE), lambda p: (0, 0)),
        pl.BlockSpec((NC, 1, B2, D), lambda p: (0, p, 0, 0)),
        pl.BlockSpec((D, D), lambda p: (0, 0)),
        pl.BlockSpec((D, D), lambda p: (0, 0)),
        pl.BlockSpec((1, D), lambda p: (0, 0)),
        pl.BlockSpec((1, D, HID), lambda p: (jnp.minimum(p, NARG), 0, 0)),
        pl.BlockSpec((1, 6, HID), lambda p: (jnp.minimum(p, NARG), 0, 0)),
        pl.BlockSpec((1, NOISE, HID), lambda p: (jnp.minimum(p, NARG), 0, 0)),
        pl.BlockSpec((1, 1, HID), lambda p: (jnp.minimum(p, NARG), 0, 0)),
        pl.BlockSpec((1, HID, D), lambda p: (jnp.minimum(p, NARG), 0, 0)),
        pl.BlockSpec((1, 1, D), lambda p: (jnp.minimum(p, NARG), 0, 0)),
    ],
    out_specs=pl.BlockSpec((1, B, D), lambda p: (p, 0, 0)),
    out_shape=jax.ShapeDtypeStruct((NPG, B, D), jnp.float32),
)


def kernel(x, edge_index, ptr, noise, Ws, Wn, bg, aW1, ab1, aW2, ab2,
           sW1, sb1, sW2, sb2):
    del ptr
    xsel = x[:, :D]
    xp = xsel.reshape(B, NPG, D).transpose(1, 0, 2)

    pad_iota = jnp.arange(EP - E, dtype=jnp.int32)
    pad = jnp.stack([pad_iota % N, N + pad_iota % (NPAD - N)])
    eip = jnp.concatenate([edge_index, pad], axis=1)
    zeros = jnp.zeros((RPT, D), jnp.float32)

    aggp = _make_sc_agg()(xsel, eip, zeros)
    aggp = aggp.reshape(NC, NPG, B2, D)

    NU = NARG + 1
    W1 = jnp.concatenate([aW1, sW1[None]])
    W1a = W1[:, :D, :]
    W1b = W1[:, D:D + 6, :]
    W1n = W1[:, D + 6:, :]
    b1 = jnp.concatenate([ab1, sb1[None]])
    W2 = jnp.concatenate([aW2, sW2[None]])
    b2 = jnp.concatenate([ab2, sb2[None]])

    outp = _tc_mlp(xp, noise, aggp, Ws, Wn, bg.reshape(1, D),
                   W1a, W1b, W1n, b1.reshape(NU, 1, HID),
                   W2, b2.reshape(NU, 1, D))
    return outp.transpose(1, 0, 2).reshape(N, D)
